# initial kernel scaffold (unmeasured)
import jax
import jax.numpy as jnp
from jax import lax
from jax.experimental import pallas as pl
from jax.experimental.pallas import tpu as pltpu


def kernel(
    x,
):
    def body(*refs):
        pass

    out_shape = jax.ShapeDtypeStruct(..., jnp.float32)
    return pl.pallas_call(body, out_shape=out_shape)(...)


# baseline (device time: 1587170 ns/iter reference)
import jax
import jax.numpy as jnp
from jax import lax
from jax.experimental import pallas as pl
from jax.experimental.pallas import tpu as pltpu

CHUNK = 2048


def kernel(x):
    m, n = x.shape
    n_chunks = m // CHUNK

    def body(x_ref, out_ref, recv_ref, send_sem, recv_sem, credit_sem):
        i = pl.program_id(0)
        my_x = lax.axis_index("x")
        my_y = lax.axis_index("y")
        my_z = lax.axis_index("z")
        partner = (my_x, my_y, 1 - my_z)

        @pl.when(i == 0)
        def _():
            barrier = pltpu.get_barrier_semaphore()
            pl.semaphore_signal(
                barrier,
                inc=1,
                device_id=partner,
                device_id_type=pl.DeviceIdType.MESH,
            )
            pl.semaphore_wait(barrier, 1)

        @pl.when(i > 0)
        def _():
            pl.semaphore_wait(credit_sem, 1)

        rdma = pltpu.make_async_remote_copy(
            src_ref=x_ref,
            dst_ref=recv_ref,
            send_sem=send_sem,
            recv_sem=recv_sem,
            device_id=partner,
            device_id_type=pl.DeviceIdType.MESH,
        )
        rdma.start()
        rdma.wait()

        out_ref[...] = x_ref[...] + recv_ref[...]

        @pl.when(i < n_chunks - 1)
        def _():
            pl.semaphore_signal(
                credit_sem,
                inc=1,
                device_id=partner,
                device_id_type=pl.DeviceIdType.MESH,
            )

    return pl.pallas_call(
        body,
        grid=(n_chunks,),
        in_specs=[pl.BlockSpec((CHUNK, n), lambda i: (i, 0))],
        out_specs=pl.BlockSpec((CHUNK, n), lambda i: (i, 0)),
        out_shape=jax.ShapeDtypeStruct((m, n), x.dtype),
        scratch_shapes=[
            pltpu.VMEM((CHUNK, n), x.dtype),
            pltpu.SemaphoreType.DMA,
            pltpu.SemaphoreType.DMA,
            pltpu.SemaphoreType.REGULAR,
        ],
        compiler_params=pltpu.CompilerParams(
            collective_id=0,
            vmem_limit_bytes=100 * 1024 * 1024,
        ),
    )(x)


# device time: 1587109 ns/iter; 1.0000x vs baseline; 1.0000x over previous
import jax
import jax.numpy as jnp
from jax import lax
from jax.experimental import pallas as pl
from jax.experimental.pallas import tpu as pltpu

CHUNK = 2048


def kernel(x):
    m, n = x.shape
    n_chunks = m // CHUNK

    def body(x_ref, out_ref, recv_ref, send_sem, recv_sem, credit_sem):
        i = pl.program_id(0)
        my_x = lax.axis_index("x")
        my_y = lax.axis_index("y")
        my_z = lax.axis_index("z")
        partner = (my_x, my_y, 1 - my_z)

        @pl.when(i == 0)
        def _():
            barrier = pltpu.get_barrier_semaphore()
            pl.semaphore_signal(
                barrier,
                inc=1,
                device_id=partner,
                device_id_type=pl.DeviceIdType.MESH,
            )
            pl.semaphore_wait(barrier, 1)

        @pl.when(i > 0)
        def _():
            pl.semaphore_wait(credit_sem, 1)

        half = CHUNK // 2
        rdma_a = pltpu.make_async_remote_copy(
            src_ref=x_ref.at[pl.ds(0, half), :],
            dst_ref=recv_ref.at[pl.ds(0, half), :],
            send_sem=send_sem.at[0],
            recv_sem=recv_sem.at[0],
            device_id=partner,
            device_id_type=pl.DeviceIdType.MESH,
        )
        rdma_b = pltpu.make_async_remote_copy(
            src_ref=x_ref.at[pl.ds(half, half), :],
            dst_ref=recv_ref.at[pl.ds(half, half), :],
            send_sem=send_sem.at[1],
            recv_sem=recv_sem.at[1],
            device_id=partner,
            device_id_type=pl.DeviceIdType.MESH,
        )
        rdma_a.start()
        rdma_b.start()
        rdma_a.wait()
        rdma_b.wait()

        out_ref[...] = x_ref[...] + recv_ref[...]

        @pl.when(i < n_chunks - 1)
        def _():
            pl.semaphore_signal(
                credit_sem,
                inc=1,
                device_id=partner,
                device_id_type=pl.DeviceIdType.MESH,
            )

    return pl.pallas_call(
        body,
        grid=(n_chunks,),
        in_specs=[pl.BlockSpec((CHUNK, n), lambda i: (i, 0))],
        out_specs=pl.BlockSpec((CHUNK, n), lambda i: (i, 0)),
        out_shape=jax.ShapeDtypeStruct((m, n), x.dtype),
        scratch_shapes=[
            pltpu.VMEM((CHUNK, n), x.dtype),
            pltpu.SemaphoreType.DMA((2,)),
            pltpu.SemaphoreType.DMA((2,)),
            pltpu.SemaphoreType.REGULAR,
        ],
        compiler_params=pltpu.CompilerParams(
            collective_id=0,
            vmem_limit_bytes=100 * 1024 * 1024,
        ),
    )(x)


# device time: 1553122 ns/iter; 1.0219x vs baseline; 1.0219x over previous
import jax
import jax.numpy as jnp
from jax import lax
from jax.experimental import pallas as pl
from jax.experimental.pallas import tpu as pltpu

CHUNK = 1024


def kernel(x):
    m, n = x.shape
    n_chunks = m // CHUNK

    def body(x_send_ref, x_add_ref, out_ref, recv_ref, send_sems, recv_sems,
             credit_sem):
        i = pl.program_id(0)
        my_x = lax.axis_index("x")
        my_y = lax.axis_index("y")
        my_z = lax.axis_index("z")
        partner = (my_x, my_y, 1 - my_z)
        send_slot = i % 2
        recv_slot = (i - 1) % 2

        @pl.when(i == 0)
        def _():
            barrier = pltpu.get_barrier_semaphore()
            pl.semaphore_signal(
                barrier,
                inc=1,
                device_id=partner,
                device_id_type=pl.DeviceIdType.MESH,
            )
            pl.semaphore_wait(barrier, 1)

        @pl.when((i >= 2) & (i < n_chunks))
        def _():
            pl.semaphore_wait(credit_sem, 1)

        @pl.when(i < n_chunks)
        def _():
            send = pltpu.make_async_remote_copy(
                src_ref=x_send_ref,
                dst_ref=recv_ref.at[send_slot],
                send_sem=send_sems.at[send_slot],
                recv_sem=recv_sems.at[send_slot],
                device_id=partner,
                device_id_type=pl.DeviceIdType.MESH,
            )
            send.start()

        @pl.when(i > 0)
        def _():
            recv = pltpu.make_async_remote_copy(
                src_ref=x_send_ref,
                dst_ref=recv_ref.at[recv_slot],
                send_sem=send_sems.at[recv_slot],
                recv_sem=recv_sems.at[recv_slot],
                device_id=partner,
                device_id_type=pl.DeviceIdType.MESH,
            )
            recv.wait_recv()
            out_ref[...] = x_add_ref[...] + recv_ref[recv_slot]

        @pl.when((i >= 1) & (i <= n_chunks - 2))
        def _():
            pl.semaphore_signal(
                credit_sem,
                inc=1,
                device_id=partner,
                device_id_type=pl.DeviceIdType.MESH,
            )

        @pl.when(i < n_chunks)
        def _():
            send = pltpu.make_async_remote_copy(
                src_ref=x_send_ref,
                dst_ref=recv_ref.at[send_slot],
                send_sem=send_sems.at[send_slot],
                recv_sem=recv_sems.at[send_slot],
                device_id=partner,
                device_id_type=pl.DeviceIdType.MESH,
            )
            send.wait_send()

    last = n_chunks - 1
    return pl.pallas_call(
        body,
        grid=(n_chunks + 1,),
        in_specs=[
            pl.BlockSpec((CHUNK, n), lambda i: (jnp.minimum(i, last), 0)),
            pl.BlockSpec((CHUNK, n), lambda i: (jnp.maximum(i - 1, 0), 0)),
        ],
        out_specs=pl.BlockSpec((CHUNK, n), lambda i: (jnp.maximum(i - 1, 0), 0)),
        out_shape=jax.ShapeDtypeStruct((m, n), x.dtype),
        scratch_shapes=[
            pltpu.VMEM((2, CHUNK, n), x.dtype),
            pltpu.SemaphoreType.DMA((2,)),
            pltpu.SemaphoreType.DMA((2,)),
            pltpu.SemaphoreType.REGULAR,
        ],
        compiler_params=pltpu.CompilerParams(
            collective_id=0,
            vmem_limit_bytes=100 * 1024 * 1024,
        ),
    )(x, x)
